# baseline (speedup 1.0000x reference)
import functools

import jax
import jax.numpy as jnp
from jax.experimental import pallas as pl
from jax.experimental.pallas import tpu as pltpu

B = 64
DIM = 768
HID = 3072
E = 8
HW = 24 * 24


def _pool_kernel(x_ref, o_ref):
    o_ref[0] = jnp.sum(x_ref[...], axis=-1) * (1.0 / HW)


def _moe_kernel(xp_ref, gamma_ref, beta_ref, wg_ref, bg_ref,
                w1_ref, b1_ref, w2_ref, b2_ref,
                ffn_ref, aux_ref,
                xn_ref, coef_ref):
    e = pl.program_id(0)

    @pl.when(e == 0)
    def _gating():
        xp = xp_ref[...]
        mu = jnp.mean(xp, axis=-1, keepdims=True)
        var = jnp.mean((xp - mu) ** 2, axis=-1, keepdims=True)
        xn = (xp - mu) * jax.lax.rsqrt(var + 1e-5) * gamma_ref[...] + beta_ref[...]
        xn_ref[...] = xn
        logits = jax.lax.dot_general(
            xn, wg_ref[...], (((1,), (1,)), ((), ())),
            preferred_element_type=jnp.float32,
            precision=jax.lax.Precision.HIGHEST) + bg_ref[...]
        io = jax.lax.broadcasted_iota(jnp.int32, (B, E), 1)
        v1 = jnp.max(logits, axis=-1, keepdims=True)
        idx1 = jnp.min(jnp.where(logits == v1, io, E), axis=-1, keepdims=True)
        m1 = io == idx1
        logits_m = jnp.where(m1, -jnp.inf, logits)
        v2 = jnp.max(logits_m, axis=-1, keepdims=True)
        idx2 = jnp.min(jnp.where(logits_m == v2, io, E), axis=-1, keepdims=True)
        m2 = io == idx2
        z = jnp.exp(v2 - v1)
        w_a = 1.0 / (1.0 + z)
        w_b = z / (1.0 + z)
        coef_ref[...] = w_a * m1.astype(jnp.float32) + w_b * m2.astype(jnp.float32)
        p = jnp.exp(logits - v1)
        p = p / jnp.sum(p, axis=-1, keepdims=True)
        imp = jnp.sum(p, axis=0, keepdims=True)
        mi = jnp.mean(imp, axis=-1, keepdims=True)
        vi = jnp.mean((imp - mi) ** 2, axis=-1, keepdims=True)
        load = jnp.sum(m1.astype(jnp.float32) + m2.astype(jnp.float32),
                       axis=0, keepdims=True)
        ml = jnp.mean(load, axis=-1, keepdims=True)
        vl = jnp.mean((load - ml) ** 2, axis=-1, keepdims=True)
        aux_ref[...] = vi / (mi * mi + 1e-10) + vl / (ml * ml + 1e-10)
        ffn_ref[...] = jnp.zeros_like(ffn_ref)

    xn = xn_ref[...]
    h = jax.lax.dot_general(
        xn, w1_ref[0], (((1,), (0,)), ((), ())),
        preferred_element_type=jnp.float32,
        precision=jax.lax.Precision.HIGHEST) + b1_ref[0]
    h = h * jax.nn.sigmoid(h)
    o = jax.lax.dot_general(
        h, w2_ref[0], (((1,), (0,)), ((), ())),
        preferred_element_type=jnp.float32,
        precision=jax.lax.Precision.HIGHEST) + b2_ref[0]
    io = jax.lax.broadcasted_iota(jnp.int32, (B, E), 1)
    c = jnp.sum(jnp.where(io == e, coef_ref[...], 0.0), axis=-1, keepdims=True)
    ffn_ref[...] += c * o


def _add_kernel(x_ref, ffn_ref, o_ref):
    o_ref[...] = x_ref[...] + ffn_ref[0][:, :, None]


@functools.partial(jax.jit, static_argnames=("interpret",))
def kernel(x, gamma, beta, Wg, bg, W1, b1, W2, b2, interpret=False):
    x4 = x.reshape(B, DIM, HW)
    bb = 4
    x_pool = pl.pallas_call(
        _pool_kernel,
        grid=(B // bb,),
        in_specs=[pl.BlockSpec((bb, DIM, HW), lambda i: (i, 0, 0))],
        out_specs=pl.BlockSpec((1, bb, DIM), lambda i: (i, 0, 0)),
        out_shape=jax.ShapeDtypeStruct((B // bb, bb, DIM), jnp.float32),
        interpret=interpret,
    )(x4).reshape(B, DIM)

    ffn, aux = pl.pallas_call(
        _moe_kernel,
        grid=(E,),
        in_specs=[
            pl.BlockSpec((B, DIM), lambda e: (0, 0)),
            pl.BlockSpec((1, DIM), lambda e: (0, 0)),
            pl.BlockSpec((1, DIM), lambda e: (0, 0)),
            pl.BlockSpec((E, DIM), lambda e: (0, 0)),
            pl.BlockSpec((1, E), lambda e: (0, 0)),
            pl.BlockSpec((1, DIM, HID), lambda e: (e, 0, 0)),
            pl.BlockSpec((1, 1, HID), lambda e: (e, 0, 0)),
            pl.BlockSpec((1, HID, DIM), lambda e: (e, 0, 0)),
            pl.BlockSpec((1, 1, DIM), lambda e: (e, 0, 0)),
        ],
        out_specs=[
            pl.BlockSpec((B, DIM), lambda e: (0, 0)),
            pl.BlockSpec((1, 1), lambda e: (0, 0)),
        ],
        out_shape=[
            jax.ShapeDtypeStruct((B, DIM), jnp.float32),
            jax.ShapeDtypeStruct((1, 1), jnp.float32),
        ],
        scratch_shapes=[
            pltpu.VMEM((B, DIM), jnp.float32),
            pltpu.VMEM((B, E), jnp.float32),
        ],
        interpret=interpret,
    )(x_pool, gamma.reshape(1, DIM), beta.reshape(1, DIM), Wg,
      bg.reshape(1, E), W1, b1.reshape(E, 1, HID), W2, b2.reshape(E, 1, DIM))

    out = pl.pallas_call(
        _add_kernel,
        grid=(B // bb,),
        in_specs=[
            pl.BlockSpec((bb, DIM, HW), lambda i: (i, 0, 0)),
            pl.BlockSpec((1, bb, DIM), lambda i: (i, 0, 0)),
        ],
        out_specs=pl.BlockSpec((bb, DIM, HW), lambda i: (i, 0, 0)),
        out_shape=jax.ShapeDtypeStruct((B, DIM, HW), jnp.float32),
        interpret=interpret,
    )(x4, ffn.reshape(B // bb, bb, DIM))

    return out.reshape(x.shape), aux[0, 0]


# speedup vs baseline: 1.1125x; 1.1125x over previous
import functools

import jax
import jax.numpy as jnp
from jax.experimental import pallas as pl
from jax.experimental.pallas import tpu as pltpu

B = 64
DIM = 768
HID = 3072
E = 8
HW = 24 * 24


def _pool_kernel(x_ref, o_ref):
    o_ref[0] = jnp.sum(x_ref[...], axis=-1) * (1.0 / HW)


def _moe_kernel(xp_ref, gamma_ref, beta_ref, wg_ref, bg_ref,
                w1_ref, b1_ref, w2_ref, b2_ref,
                ffn_ref, aux_ref,
                xn_ref, coef_ref):
    e = pl.program_id(0)

    @pl.when(e == 0)
    def _gating():
        xp = xp_ref[...]
        mu = jnp.mean(xp, axis=-1, keepdims=True)
        var = jnp.mean((xp - mu) ** 2, axis=-1, keepdims=True)
        xn = (xp - mu) * jax.lax.rsqrt(var + 1e-5) * gamma_ref[...] + beta_ref[...]
        xn_ref[...] = xn
        logits = jax.lax.dot_general(
            xn, wg_ref[...], (((1,), (1,)), ((), ())),
            preferred_element_type=jnp.float32,
            precision=jax.lax.Precision.HIGHEST) + bg_ref[...]
        io = jax.lax.broadcasted_iota(jnp.int32, (B, E), 1)
        v1 = jnp.max(logits, axis=-1, keepdims=True)
        idx1 = jnp.min(jnp.where(logits == v1, io, E), axis=-1, keepdims=True)
        m1 = io == idx1
        logits_m = jnp.where(m1, -jnp.inf, logits)
        v2 = jnp.max(logits_m, axis=-1, keepdims=True)
        idx2 = jnp.min(jnp.where(logits_m == v2, io, E), axis=-1, keepdims=True)
        m2 = io == idx2
        z = jnp.exp(v2 - v1)
        w_a = 1.0 / (1.0 + z)
        w_b = z / (1.0 + z)
        coef_ref[...] = w_a * m1.astype(jnp.float32) + w_b * m2.astype(jnp.float32)
        p = jnp.exp(logits - v1)
        p = p / jnp.sum(p, axis=-1, keepdims=True)
        imp = jnp.sum(p, axis=0, keepdims=True)
        mi = jnp.mean(imp, axis=-1, keepdims=True)
        vi = jnp.mean((imp - mi) ** 2, axis=-1, keepdims=True)
        load = jnp.sum(m1.astype(jnp.float32) + m2.astype(jnp.float32),
                       axis=0, keepdims=True)
        ml = jnp.mean(load, axis=-1, keepdims=True)
        vl = jnp.mean((load - ml) ** 2, axis=-1, keepdims=True)
        aux_ref[...] = vi / (mi * mi + 1e-10) + vl / (ml * ml + 1e-10)
        ffn_ref[...] = jnp.zeros_like(ffn_ref)

    xn = xn_ref[...].astype(jnp.bfloat16)
    h = jax.lax.dot_general(
        xn, w1_ref[0].astype(jnp.bfloat16), (((1,), (0,)), ((), ())),
        preferred_element_type=jnp.float32) + b1_ref[0]
    h = h * jax.nn.sigmoid(h)
    o = jax.lax.dot_general(
        h.astype(jnp.bfloat16), w2_ref[0].astype(jnp.bfloat16),
        (((1,), (0,)), ((), ())),
        preferred_element_type=jnp.float32) + b2_ref[0]
    io = jax.lax.broadcasted_iota(jnp.int32, (B, E), 1)
    c = jnp.sum(jnp.where(io == e, coef_ref[...], 0.0), axis=-1, keepdims=True)
    ffn_ref[...] += c * o


def _add_kernel(x_ref, ffn_ref, o_ref):
    o_ref[...] = x_ref[...] + ffn_ref[0][:, :, None]


@functools.partial(jax.jit, static_argnames=("interpret",))
def kernel(x, gamma, beta, Wg, bg, W1, b1, W2, b2, interpret=False):
    x4 = x.reshape(B, DIM, HW)
    bb = 4
    x_pool = pl.pallas_call(
        _pool_kernel,
        grid=(B // bb,),
        in_specs=[pl.BlockSpec((bb, DIM, HW), lambda i: (i, 0, 0))],
        out_specs=pl.BlockSpec((1, bb, DIM), lambda i: (i, 0, 0)),
        out_shape=jax.ShapeDtypeStruct((B // bb, bb, DIM), jnp.float32),
        interpret=interpret,
    )(x4).reshape(B, DIM)

    ffn, aux = pl.pallas_call(
        _moe_kernel,
        grid=(E,),
        in_specs=[
            pl.BlockSpec((B, DIM), lambda e: (0, 0)),
            pl.BlockSpec((1, DIM), lambda e: (0, 0)),
            pl.BlockSpec((1, DIM), lambda e: (0, 0)),
            pl.BlockSpec((E, DIM), lambda e: (0, 0)),
            pl.BlockSpec((1, E), lambda e: (0, 0)),
            pl.BlockSpec((1, DIM, HID), lambda e: (e, 0, 0)),
            pl.BlockSpec((1, 1, HID), lambda e: (e, 0, 0)),
            pl.BlockSpec((1, HID, DIM), lambda e: (e, 0, 0)),
            pl.BlockSpec((1, 1, DIM), lambda e: (e, 0, 0)),
        ],
        out_specs=[
            pl.BlockSpec((B, DIM), lambda e: (0, 0)),
            pl.BlockSpec((1, 1), lambda e: (0, 0)),
        ],
        out_shape=[
            jax.ShapeDtypeStruct((B, DIM), jnp.float32),
            jax.ShapeDtypeStruct((1, 1), jnp.float32),
        ],
        scratch_shapes=[
            pltpu.VMEM((B, DIM), jnp.float32),
            pltpu.VMEM((B, E), jnp.float32),
        ],
        interpret=interpret,
    )(x_pool, gamma.reshape(1, DIM), beta.reshape(1, DIM), Wg,
      bg.reshape(1, E), W1, b1.reshape(E, 1, HID), W2, b2.reshape(E, 1, DIM))

    out = pl.pallas_call(
        _add_kernel,
        grid=(B // bb,),
        in_specs=[
            pl.BlockSpec((bb, DIM, HW), lambda i: (i, 0, 0)),
            pl.BlockSpec((1, bb, DIM), lambda i: (i, 0, 0)),
        ],
        out_specs=pl.BlockSpec((bb, DIM, HW), lambda i: (i, 0, 0)),
        out_shape=jax.ShapeDtypeStruct((B, DIM, HW), jnp.float32),
        interpret=interpret,
    )(x4, ffn.reshape(B // bb, bb, DIM))

    return out.reshape(x.shape), aux[0, 0]
